# baseline (speedup 1.0000x reference)
import functools

import jax
import jax.numpy as jnp
from jax import lax
from jax.experimental import pallas as pl
from jax.experimental.pallas import tpu as pltpu
from jax.experimental.pallas import tpu_sc as plsc

_B = 16384
_MAX_OUT = 200
_EMB = 64
_STATE = 128
_HID = 256
_NEG = -99999.0

_NC = 2
_NS = 16
_NW = _NC * _NS
_LANES = 16

_BPW = _B // _NW
_CB = 8
_NCHUNK = _BPW // _CB
_SUB = 100
_SUBS = _CB * _MAX_OUT // _SUB
_JB = 7
_SC_PAD = _CB * _MAX_OUT + _LANES

def _sc_gather_pairs_body(idx_hbm, table_hbm, out_hbm, ids_v, rows_v, sem):
    wid = lax.axis_index("s") * _NC + lax.axis_index("c")
    pltpu.sync_copy(idx_hbm.at[pl.ds(wid * 8, 8)], ids_v)
    for sub in range(8):
        pltpu.async_copy(
            table_hbm.at[ids_v.at[sub]],
            rows_v.at[pl.ds(sub * 128, 128)],
            sem,
        ).wait()
    pltpu.sync_copy(rows_v, out_hbm.at[pl.ds(wid * 1024, 1024)])


def _sc_gather_pairs(pair_idx, table):
    mesh = plsc.VectorSubcoreMesh(core_axis_name="c", subcore_axis_name="s")
    call = functools.partial(
        pl.kernel,
        mesh=mesh,
        compiler_params=pltpu.CompilerParams(use_tc_tiling_on_sc=False, needs_layout_passes=False),
        out_type=jax.ShapeDtypeStruct((2 * _B, _EMB), jnp.float32),
        scratch_types=[
            pltpu.VMEM((8, 128), jnp.int32),
            pltpu.VMEM((1024, _EMB), jnp.float32),
            pltpu.SemaphoreType.DMA,
        ],
    )(_sc_gather_pairs_body)
    return call(pair_idx, table)


def _sc_score_body(ids_hbm, table_hbm, outv_hbm, scores_hbm,
                   ids_v, rows_v, outv_v, scores_v, sem):
    wid = lax.axis_index("s") * _NC + lax.axis_index("c")
    pltpu.sync_copy(outv_hbm.at[pl.ds(wid * _BPW, _BPW)], outv_v)
    lane = lax.iota(jnp.int32, _LANES)
    idrows_pw = _BPW * _MAX_OUT // _SUB

    def chunk_body(c, carry):
        pltpu.sync_copy(
            ids_hbm.at[pl.ds(wid * idrows_pw + c * _SUBS, _SUBS)], ids_v)

        def sub_body(sub, carry2):
            pltpu.async_copy(table_hbm.at[ids_v.at[sub]], rows_v, sem).wait()
            b_local = c * _CB + sub // 2
            ov = [outv_v[b_local, pl.ds(g * _LANES, _LANES)]
                  for g in range(_EMB // _LANES)]

            def jb_body(jb, carry3):
                row_ids = jnp.minimum(jb * _LANES + lane, _SUB - 1)
                acc = jnp.zeros((_LANES,), jnp.float32)
                for k in range(_EMB):
                    vals = plsc.load_gather(
                        rows_v, [row_ids, jnp.full((_LANES,), k, jnp.int32)])
                    acc = acc + vals * ov[k // _LANES][k % _LANES]
                scores_v[pl.ds(sub * _SUB + jb * _LANES, _LANES)] = acc
                return carry3

            lax.fori_loop(0, _JB, jb_body, 0)
            return carry2

        lax.fori_loop(0, _SUBS, sub_body, 0)
        pltpu.sync_copy(
            scores_v.at[pl.ds(0, _CB * _MAX_OUT)],
            scores_hbm.at[pl.ds((wid * _BPW + c * _CB) * _MAX_OUT,
                                _CB * _MAX_OUT)],
        )
        return carry

    lax.fori_loop(0, _NCHUNK, chunk_body, 0)


def _sc_score(flat_ids, table, outv):
    mesh = plsc.VectorSubcoreMesh(core_axis_name="c", subcore_axis_name="s")
    call = functools.partial(
        pl.kernel,
        mesh=mesh,
        compiler_params=pltpu.CompilerParams(use_tc_tiling_on_sc=False, needs_layout_passes=False),
        out_type=jax.ShapeDtypeStruct((_B * _MAX_OUT,), jnp.float32),
        scratch_types=[
            pltpu.VMEM((_SUBS, _SUB), jnp.int32),
            pltpu.VMEM((_SUB, _EMB), jnp.float32),
            pltpu.VMEM((_BPW, _EMB), jnp.float32),
            pltpu.VMEM((_SC_PAD,), jnp.float32),
            pltpu.SemaphoreType.DMA,
        ],
    )(_sc_score_body)
    return call(flat_ids, table, outv)


def _lstm_mlp_body(prev_ref, h_ref, c_ref, q_ref, wih_ref, whh_ref, b_ref,
                   l1_ref, l1b_ref, l2_ref, l2b_ref, out_ref):
    gates = (jnp.dot(prev_ref[...], wih_ref[...],
                     preferred_element_type=jnp.float32)
             + jnp.dot(h_ref[...], whh_ref[...],
                       preferred_element_type=jnp.float32)
             + b_ref[...])
    i_g = jax.nn.sigmoid(gates[:, :_STATE])
    f_g = jax.nn.sigmoid(gates[:, _STATE:2 * _STATE])
    g_g = jnp.tanh(gates[:, 2 * _STATE:3 * _STATE])
    o_g = jax.nn.sigmoid(gates[:, 3 * _STATE:])
    c_new = f_g * c_ref[...] + i_g * g_g
    h_new = o_g * jnp.tanh(c_new)
    sq = jnp.concatenate([h_new, q_ref[...]], axis=1)
    hid = jnp.maximum(
        jnp.dot(sq, l1_ref[...], preferred_element_type=jnp.float32)
        + l1b_ref[...], 0.0)
    out_ref[...] = jnp.maximum(
        jnp.dot(hid, l2_ref[...], preferred_element_type=jnp.float32)
        + l2b_ref[...], 0.0)


def _sample_body(flag_ref, prelim_ref, rel_ref, ent_ref, gum_ref, rnd_ref,
                 loss_ref, logits_ref, aid_ref, nent_ref, crel_ref):
    flag = flag_ref[0, 0]
    ms = jnp.where(flag != 0, rnd_ref[...], prelim_ref[...])
    ent = ent_ref[...]
    scores = jnp.where(ent == 0, jnp.float32(_NEG), ms)
    m = jnp.max(scores, axis=1, keepdims=True)
    shifted = scores - m
    lse = jnp.log(jnp.sum(jnp.exp(shifted), axis=1, keepdims=True))
    logits = shifted - lse
    z = logits + gum_ref[...]
    zmax = jnp.max(z, axis=1, keepdims=True)
    iota = lax.broadcasted_iota(jnp.int32, z.shape, 1)
    aid = jnp.min(jnp.where(z == zmax, iota, _MAX_OUT), axis=1)
    sel = iota == aid[:, None]
    loss = -jnp.sum(jnp.where(sel, logits, 0.0), axis=1)
    crel = jnp.sum(jnp.where(sel, rel_ref[...], 0), axis=1)
    nent = jnp.sum(jnp.where(sel, ent, 0), axis=1)
    logits_ref[...] = logits
    loss_ref[...] = loss[:, None]
    aid_ref[...] = aid[:, None]
    nent_ref[...] = nent[:, None]
    crel_ref[...] = crel[:, None]


_BB = 2048
_BD = 2048


def _tc_lstm_mlp(prev_emb, q_emb, state_h, state_c, wih_t, whh_t, bias,
                 l1_t, l1b, l2_t, l2b):
    nb = _B // _BB
    row = lambda i: (i, 0)
    rep = lambda i: (0, 0)
    return pl.pallas_call(
        _lstm_mlp_body,
        grid=(nb,),
        in_specs=[
            pl.BlockSpec((_BB, _EMB), row),
            pl.BlockSpec((_BB, _STATE), row),
            pl.BlockSpec((_BB, _STATE), row),
            pl.BlockSpec((_BB, _EMB), row),
            pl.BlockSpec((_EMB, 4 * _STATE), rep),
            pl.BlockSpec((_STATE, 4 * _STATE), rep),
            pl.BlockSpec((1, 4 * _STATE), rep),
            pl.BlockSpec((_STATE + _EMB, _HID), rep),
            pl.BlockSpec((1, _HID), rep),
            pl.BlockSpec((_HID, _EMB), rep),
            pl.BlockSpec((1, _EMB), rep),
        ],
        out_specs=pl.BlockSpec((_BB, _EMB), row),
        out_shape=jax.ShapeDtypeStruct((_B, _EMB), jnp.float32),
    )(prev_emb, state_h, state_c, q_emb, wih_t, whh_t, bias,
      l1_t, l1b, l2_t, l2b)


def _tc_sample(flag, prelim, rel_ids, ent_ids, gum, rnd):
    nb = _B // _BD
    row = lambda i: (i, 0)
    return pl.pallas_call(
        _sample_body,
        grid=(nb,),
        in_specs=[
            pl.BlockSpec(memory_space=pltpu.SMEM),
            pl.BlockSpec((_BD, _MAX_OUT), row),
            pl.BlockSpec((_BD, _MAX_OUT), row),
            pl.BlockSpec((_BD, _MAX_OUT), row),
            pl.BlockSpec((_BD, _MAX_OUT), row),
            pl.BlockSpec((_BD, _MAX_OUT), row),
        ],
        out_specs=[
            pl.BlockSpec((_BD, 1), row),
            pl.BlockSpec((_BD, _MAX_OUT), row),
            pl.BlockSpec((_BD, 1), row),
            pl.BlockSpec((_BD, 1), row),
            pl.BlockSpec((_BD, 1), row),
        ],
        out_shape=[
            jax.ShapeDtypeStruct((_B, 1), jnp.float32),
            jax.ShapeDtypeStruct((_B, _MAX_OUT), jnp.float32),
            jax.ShapeDtypeStruct((_B, 1), jnp.int32),
            jax.ShapeDtypeStruct((_B, 1), jnp.int32),
            jax.ShapeDtypeStruct((_B, 1), jnp.int32),
        ],
    )(flag, prelim, rel_ids, ent_ids, gum, rnd)


def kernel(prev_relation, current_entity, actions_id, queries, random,
           item_embedding, w_ih, w_hh, b_ih, b_hh, mlp_l1_w, mlp_l1_b,
           mlp_l2_w, mlp_l2_b, state_h, state_c):
    rel_ids = actions_id[:, :, 0]
    ent_ids = actions_id[:, :, 1]

    pair_idx = jnp.concatenate(
        [prev_relation, queries]).astype(jnp.int32).reshape(_NW * 8, 128)
    pair_emb = _sc_gather_pairs(pair_idx, item_embedding)
    prev_emb = pair_emb[:_B]
    q_emb = pair_emb[_B:]

    outv = _tc_lstm_mlp(
        prev_emb, q_emb, state_h, state_c,
        w_ih.T, w_hh.T, (b_ih + b_hh).reshape(1, 4 * _STATE),
        mlp_l1_w.T, mlp_l1_b.reshape(1, _HID),
        mlp_l2_w.T, mlp_l2_b.reshape(1, _EMB))

    flat_ids = rel_ids.astype(jnp.int32).reshape(_B * _MAX_OUT // _SUB, _SUB)
    model_scores = _sc_score(flat_ids, item_embedding, outv)
    model_scores = model_scores.reshape(_B, _MAX_OUT)

    gum = jax.random.gumbel(jax.random.key(42), (_B, _MAX_OUT), jnp.float32)
    rnd = jax.random.normal(jax.random.key(7), (_B, _MAX_OUT), jnp.float32)
    flag = jnp.asarray(random, jnp.int32).reshape(1, 1)

    loss, logits, aid, nent, crel = _tc_sample(
        flag, model_scores, rel_ids, ent_ids, gum, rnd)
    return (loss[:, 0], logits, aid[:, 0], nent[:, 0], crel[:, 0])


# speedup vs baseline: 1.1127x; 1.1127x over previous
import functools

import jax
import jax.numpy as jnp
from jax import lax
from jax.experimental import pallas as pl
from jax.experimental.pallas import tpu as pltpu
from jax.experimental.pallas import tpu_sc as plsc

_B = 16384
_MAX_OUT = 200
_EMB = 64
_STATE = 128
_HID = 256
_NEG = -99999.0

_NC = 2
_NS = 16
_NW = _NC * _NS
_LANES = 16

_BPW = _B // _NW
_CB = 8
_NCHUNK = _BPW // _CB
_SUB = 100
_SUBS = _CB * _MAX_OUT // _SUB
_JB = 7
_SC_PAD = _CB * _MAX_OUT + _LANES

def _sc_gather_pairs_body(idx_hbm, table_hbm, out_hbm, ids_v, rows_v, sem):
    wid = lax.axis_index("s") * _NC + lax.axis_index("c")
    pltpu.sync_copy(idx_hbm.at[pl.ds(wid * 8, 8)], ids_v)
    for sub in range(8):
        pltpu.async_copy(
            table_hbm.at[ids_v.at[sub]],
            rows_v.at[pl.ds(sub * 128, 128)],
            sem,
        ).wait()
    pltpu.sync_copy(rows_v, out_hbm.at[pl.ds(wid * 1024, 1024)])


def _sc_gather_pairs(pair_idx, table):
    mesh = plsc.VectorSubcoreMesh(core_axis_name="c", subcore_axis_name="s")
    call = functools.partial(
        pl.kernel,
        mesh=mesh,
        compiler_params=pltpu.CompilerParams(use_tc_tiling_on_sc=False, needs_layout_passes=False),
        out_type=jax.ShapeDtypeStruct((2 * _B, _EMB), jnp.float32),
        scratch_types=[
            pltpu.VMEM((8, 128), jnp.int32),
            pltpu.VMEM((1024, _EMB), jnp.float32),
            pltpu.SemaphoreType.DMA,
        ],
    )(_sc_gather_pairs_body)
    return call(pair_idx, table)


def _sc_score_body(ids_hbm, table_hbm, outv_hbm, scores_hbm,
                   ids_v, rows0, rows1, outv_v, scores_v, sem0, sem1):
    wid = lax.axis_index("s") * _NC + lax.axis_index("c")
    pltpu.sync_copy(outv_hbm.at[pl.ds(wid * _BPW, _BPW)], outv_v)
    lane = lax.iota(jnp.int32, _LANES)
    idrows_pw = _BPW * _MAX_OUT // _SUB

    def chunk_body(c, carry):
        pltpu.sync_copy(
            ids_hbm.at[pl.ds(wid * idrows_pw + c * _SUBS, _SUBS)], ids_v)
        pltpu.async_copy(table_hbm.at[ids_v.at[0]], rows0, sem0)

        def compute(sub, rows_v, ov):
            def jb_body(jb, carry3):
                row_ids = jnp.minimum(jb * _LANES + lane, _SUB - 1)
                accs = [jnp.zeros((_LANES,), jnp.float32) for _ in range(4)]
                for k in range(_EMB):
                    vals = plsc.load_gather(
                        rows_v, [row_ids, jnp.full((_LANES,), k, jnp.int32)])
                    accs[k % 4] = accs[k % 4] + vals * ov[k // _LANES][k % _LANES]
                scores_v[pl.ds(sub * _SUB + jb * _LANES, _LANES)] = (
                    (accs[0] + accs[1]) + (accs[2] + accs[3]))
                return carry3

            lax.fori_loop(0, _JB, jb_body, 0)

        def pair_body(p, carry2):
            sub0 = 2 * p
            b_local = c * _CB + p
            ov = [outv_v[b_local, pl.ds(g * _LANES, _LANES)]
                  for g in range(_EMB // _LANES)]
            pltpu.async_copy(table_hbm.at[ids_v.at[sub0 + 1]], rows1, sem1)
            pltpu.make_async_copy(
                table_hbm.at[ids_v.at[sub0]], rows0, sem0).wait()
            compute(sub0, rows0, ov)

            @pl.when(p < _SUBS // 2 - 1)
            def _():
                pltpu.async_copy(table_hbm.at[ids_v.at[sub0 + 2]], rows0, sem0)

            pltpu.make_async_copy(
                table_hbm.at[ids_v.at[sub0 + 1]], rows1, sem1).wait()
            compute(sub0 + 1, rows1, ov)
            return carry2

        lax.fori_loop(0, _SUBS // 2, pair_body, 0)
        pltpu.sync_copy(
            scores_v.at[pl.ds(0, _CB * _MAX_OUT)],
            scores_hbm.at[pl.ds((wid * _BPW + c * _CB) * _MAX_OUT,
                                _CB * _MAX_OUT)],
        )
        return carry

    lax.fori_loop(0, _NCHUNK, chunk_body, 0)


def _sc_score(flat_ids, table, outv):
    mesh = plsc.VectorSubcoreMesh(core_axis_name="c", subcore_axis_name="s")
    call = functools.partial(
        pl.kernel,
        mesh=mesh,
        compiler_params=pltpu.CompilerParams(use_tc_tiling_on_sc=False, needs_layout_passes=False),
        out_type=jax.ShapeDtypeStruct((_B * _MAX_OUT,), jnp.float32),
        scratch_types=[
            pltpu.VMEM((_SUBS, _SUB), jnp.int32),
            pltpu.VMEM((_SUB, _EMB), jnp.float32),
            pltpu.VMEM((_SUB, _EMB), jnp.float32),
            pltpu.VMEM((_BPW, _EMB), jnp.float32),
            pltpu.VMEM((_SC_PAD,), jnp.float32),
            pltpu.SemaphoreType.DMA,
            pltpu.SemaphoreType.DMA,
        ],
    )(_sc_score_body)
    return call(flat_ids, table, outv)


def _lstm_mlp_body(prev_ref, h_ref, c_ref, q_ref, wih_ref, whh_ref, b_ref,
                   l1_ref, l1b_ref, l2_ref, l2b_ref, out_ref):
    gates = (jnp.dot(prev_ref[...], wih_ref[...],
                     preferred_element_type=jnp.float32)
             + jnp.dot(h_ref[...], whh_ref[...],
                       preferred_element_type=jnp.float32)
             + b_ref[...])
    i_g = jax.nn.sigmoid(gates[:, :_STATE])
    f_g = jax.nn.sigmoid(gates[:, _STATE:2 * _STATE])
    g_g = jnp.tanh(gates[:, 2 * _STATE:3 * _STATE])
    o_g = jax.nn.sigmoid(gates[:, 3 * _STATE:])
    c_new = f_g * c_ref[...] + i_g * g_g
    h_new = o_g * jnp.tanh(c_new)
    sq = jnp.concatenate([h_new, q_ref[...]], axis=1)
    hid = jnp.maximum(
        jnp.dot(sq, l1_ref[...], preferred_element_type=jnp.float32)
        + l1b_ref[...], 0.0)
    out_ref[...] = jnp.maximum(
        jnp.dot(hid, l2_ref[...], preferred_element_type=jnp.float32)
        + l2b_ref[...], 0.0)


def _sample_body(flag_ref, prelim_ref, rel_ref, ent_ref, gum_ref, rnd_ref,
                 loss_ref, logits_ref, aid_ref, nent_ref, crel_ref):
    flag = flag_ref[0, 0]
    ms = jnp.where(flag != 0, rnd_ref[...], prelim_ref[...])
    ent = ent_ref[...]
    scores = jnp.where(ent == 0, jnp.float32(_NEG), ms)
    m = jnp.max(scores, axis=1, keepdims=True)
    shifted = scores - m
    lse = jnp.log(jnp.sum(jnp.exp(shifted), axis=1, keepdims=True))
    logits = shifted - lse
    z = logits + gum_ref[...]
    zmax = jnp.max(z, axis=1, keepdims=True)
    iota = lax.broadcasted_iota(jnp.int32, z.shape, 1)
    aid = jnp.min(jnp.where(z == zmax, iota, _MAX_OUT), axis=1)
    sel = iota == aid[:, None]
    loss = -jnp.sum(jnp.where(sel, logits, 0.0), axis=1)
    crel = jnp.sum(jnp.where(sel, rel_ref[...], 0), axis=1)
    nent = jnp.sum(jnp.where(sel, ent, 0), axis=1)
    logits_ref[...] = logits
    loss_ref[...] = loss[:, None]
    aid_ref[...] = aid[:, None]
    nent_ref[...] = nent[:, None]
    crel_ref[...] = crel[:, None]


_BB = 2048
_BD = 2048


def _tc_lstm_mlp(prev_emb, q_emb, state_h, state_c, wih_t, whh_t, bias,
                 l1_t, l1b, l2_t, l2b):
    nb = _B // _BB
    row = lambda i: (i, 0)
    rep = lambda i: (0, 0)
    return pl.pallas_call(
        _lstm_mlp_body,
        grid=(nb,),
        in_specs=[
            pl.BlockSpec((_BB, _EMB), row),
            pl.BlockSpec((_BB, _STATE), row),
            pl.BlockSpec((_BB, _STATE), row),
            pl.BlockSpec((_BB, _EMB), row),
            pl.BlockSpec((_EMB, 4 * _STATE), rep),
            pl.BlockSpec((_STATE, 4 * _STATE), rep),
            pl.BlockSpec((1, 4 * _STATE), rep),
            pl.BlockSpec((_STATE + _EMB, _HID), rep),
            pl.BlockSpec((1, _HID), rep),
            pl.BlockSpec((_HID, _EMB), rep),
            pl.BlockSpec((1, _EMB), rep),
        ],
        out_specs=pl.BlockSpec((_BB, _EMB), row),
        out_shape=jax.ShapeDtypeStruct((_B, _EMB), jnp.float32),
    )(prev_emb, state_h, state_c, q_emb, wih_t, whh_t, bias,
      l1_t, l1b, l2_t, l2b)


def _tc_sample(flag, prelim, rel_ids, ent_ids, gum, rnd):
    nb = _B // _BD
    row = lambda i: (i, 0)
    return pl.pallas_call(
        _sample_body,
        grid=(nb,),
        in_specs=[
            pl.BlockSpec(memory_space=pltpu.SMEM),
            pl.BlockSpec((_BD, _MAX_OUT), row),
            pl.BlockSpec((_BD, _MAX_OUT), row),
            pl.BlockSpec((_BD, _MAX_OUT), row),
            pl.BlockSpec((_BD, _MAX_OUT), row),
            pl.BlockSpec((_BD, _MAX_OUT), row),
        ],
        out_specs=[
            pl.BlockSpec((_BD, 1), row),
            pl.BlockSpec((_BD, _MAX_OUT), row),
            pl.BlockSpec((_BD, 1), row),
            pl.BlockSpec((_BD, 1), row),
            pl.BlockSpec((_BD, 1), row),
        ],
        out_shape=[
            jax.ShapeDtypeStruct((_B, 1), jnp.float32),
            jax.ShapeDtypeStruct((_B, _MAX_OUT), jnp.float32),
            jax.ShapeDtypeStruct((_B, 1), jnp.int32),
            jax.ShapeDtypeStruct((_B, 1), jnp.int32),
            jax.ShapeDtypeStruct((_B, 1), jnp.int32),
        ],
    )(flag, prelim, rel_ids, ent_ids, gum, rnd)


def kernel(prev_relation, current_entity, actions_id, queries, random,
           item_embedding, w_ih, w_hh, b_ih, b_hh, mlp_l1_w, mlp_l1_b,
           mlp_l2_w, mlp_l2_b, state_h, state_c):
    rel_ids = actions_id[:, :, 0]
    ent_ids = actions_id[:, :, 1]

    pair_idx = jnp.concatenate(
        [prev_relation, queries]).astype(jnp.int32).reshape(_NW * 8, 128)
    pair_emb = _sc_gather_pairs(pair_idx, item_embedding)
    prev_emb = pair_emb[:_B]
    q_emb = pair_emb[_B:]

    outv = _tc_lstm_mlp(
        prev_emb, q_emb, state_h, state_c,
        w_ih.T, w_hh.T, (b_ih + b_hh).reshape(1, 4 * _STATE),
        mlp_l1_w.T, mlp_l1_b.reshape(1, _HID),
        mlp_l2_w.T, mlp_l2_b.reshape(1, _EMB))

    flat_ids = rel_ids.astype(jnp.int32).reshape(_B * _MAX_OUT // _SUB, _SUB)
    model_scores = _sc_score(flat_ids, item_embedding, outv)
    model_scores = model_scores.reshape(_B, _MAX_OUT)

    gum = jax.random.gumbel(jax.random.key(42), (_B, _MAX_OUT), jnp.float32)
    rnd = jax.random.normal(jax.random.key(7), (_B, _MAX_OUT), jnp.float32)
    flag = jnp.asarray(random, jnp.int32).reshape(1, 1)

    loss, logits, aid, nent, crel = _tc_sample(
        flag, model_scores, rel_ids, ent_ids, gum, rnd)
    return (loss[:, 0], logits, aid[:, 0], nent[:, 0], crel[:, 0])


# speedup vs baseline: 2.0904x; 1.8787x over previous
import functools

import jax
import jax.numpy as jnp
from jax import lax
from jax.experimental import pallas as pl
from jax.experimental.pallas import tpu as pltpu
from jax.experimental.pallas import tpu_sc as plsc

_B = 16384
_MAX_OUT = 200
_EMB = 64
_STATE = 128
_HID = 256
_NEG = -99999.0

_NC = 2
_NS = 16
_NW = _NC * _NS
_LANES = 16

_BPW = _B // _NW
_CB = 8
_NCHUNK = _BPW // _CB
_SUB = 100
_SUBS = _CB * _MAX_OUT // _SUB
_JB = 7
_SC_PAD = _CB * _MAX_OUT + _LANES

def _sc_gather_pairs_body(idx_hbm, table_hbm, out_hbm, ids_v, rows_v, sem):
    wid = lax.axis_index("s") * _NC + lax.axis_index("c")
    pltpu.sync_copy(idx_hbm.at[pl.ds(wid * 8, 8)], ids_v)
    for sub in range(8):
        pltpu.async_copy(
            table_hbm.at[ids_v.at[sub]],
            rows_v.at[pl.ds(sub * 128, 128)],
            sem,
        ).wait()
    pltpu.sync_copy(rows_v, out_hbm.at[pl.ds(wid * 1024, 1024)])


def _sc_gather_pairs(pair_idx, table):
    mesh = plsc.VectorSubcoreMesh(core_axis_name="c", subcore_axis_name="s")
    call = functools.partial(
        pl.kernel,
        mesh=mesh,
        compiler_params=pltpu.CompilerParams(use_tc_tiling_on_sc=False, needs_layout_passes=False),
        out_type=jax.ShapeDtypeStruct((2 * _B, _EMB), jnp.float32),
        scratch_types=[
            pltpu.VMEM((8, 128), jnp.int32),
            pltpu.VMEM((1024, _EMB), jnp.float32),
            pltpu.SemaphoreType.DMA,
        ],
    )(_sc_gather_pairs_body)
    return call(pair_idx, table)


def _sc_score_body(ids_hbm, table_hbm, outv_hbm, scores_hbm,
                   ids_v, rows0, rows1, rows2, rows3, outv_v, scores_v,
                   sem_rows, sem_ids, sem_out):
    wid = lax.axis_index("s") * _NC + lax.axis_index("c")
    nsub = _BPW * _MAX_OUT // _SUB
    nblk = nsub // _SUBS
    rows = [rows0, rows1, rows2, rows3]
    pltpu.sync_copy(outv_hbm.at[pl.ds(wid * _BPW, _BPW)], outv_v)
    lane = lax.iota(jnp.int32, _LANES)
    idrows_pw = _BPW * _MAX_OUT // _SUB

    def ids_row(s):
        return ((s // _SUBS) % 2) * _SUBS + s % _SUBS

    def fire_rows(s, buf):
        pltpu.async_copy(table_hbm.at[ids_v.at[ids_row(s)]], buf, sem_rows)

    def fire_ids(blk):
        pltpu.async_copy(
            ids_hbm.at[pl.ds(wid * idrows_pw + blk * _SUBS, _SUBS)],
            ids_v.at[pl.ds((blk % 2) * _SUBS, _SUBS)], sem_ids)

    pltpu.sync_copy(ids_hbm.at[pl.ds(wid * idrows_pw, _SUBS)],
                    ids_v.at[pl.ds(0, _SUBS)])
    fire_ids(1)
    for u in range(3):
        fire_rows(u, rows[u])

    colv = [(lane + k0) & 15 for k0 in range(_LANES)]

    def compute(s, rows_v):
        par = (s // _SUBS) % 2
        base = par * _SC_PAD + (s % _SUBS) * _SUB
        ov = [outv_v[s // 2, pl.ds(g * _LANES, _LANES)]
              for g in range(_EMB // _LANES)]

        def jb_body(jb, carry3):
            row_ids = jnp.minimum(jb * _LANES + lane, _SUB - 1)
            accs = [jnp.zeros((_LANES,), jnp.float32) for _ in range(4)]
            for k in range(_EMB):
                g, k0 = k // _LANES, k % _LANES
                vals = plsc.load_gather(
                    rows_v, [row_ids, colv[k0] + g * _LANES])
                mult = ov[g].at[colv[k0]].get(mode="promise_in_bounds")
                accs[k % 4] = accs[k % 4] + vals * mult
            scores_v[pl.ds(base + jb * _LANES, _LANES)] = (
                (accs[0] + accs[1]) + (accs[2] + accs[3]))
            return carry3

        lax.fori_loop(0, _JB, jb_body, 0)

    def group_body(g, carry):
        for u in range(4):
            s = 4 * g + u
            t = s + 3
            blk = s // _SUBS

            @pl.when(jnp.logical_and(t % _SUBS == 0, t < nsub))
            def _():
                pltpu.make_async_copy(
                    ids_hbm.at[pl.ds(0, _SUBS)],
                    ids_v.at[pl.ds(0, _SUBS)], sem_ids).wait()

            pltpu.make_async_copy(
                table_hbm.at[ids_v.at[0]], rows[u], sem_rows).wait()

            @pl.when(t < nsub)
            def _():
                fire_rows(t, rows[(u + 3) % 4])

            compute(s, rows[u])

            @pl.when(jnp.logical_and(s % _SUBS == _SUBS - 3,
                                     blk + 2 < nblk))
            def _():
                fire_ids(blk + 2)

            @pl.when(s % _SUBS == _SUBS - 1)
            def _():
                @pl.when(blk >= 1)
                def _():
                    pltpu.make_async_copy(
                        scores_v.at[pl.ds(0, _SUBS * _SUB)],
                        scores_hbm.at[pl.ds(0, _SUBS * _SUB)],
                        sem_out).wait()
                pltpu.async_copy(
                    scores_v.at[pl.ds(((s // _SUBS) % 2) * _SC_PAD,
                                      _SUBS * _SUB)],
                    scores_hbm.at[pl.ds(wid * _BPW * _MAX_OUT
                                        + blk * _SUBS * _SUB, _SUBS * _SUB)],
                    sem_out)
        return carry

    lax.fori_loop(0, nsub // 4, group_body, 0)
    pltpu.make_async_copy(
        scores_v.at[pl.ds(0, _SUBS * _SUB)],
        scores_hbm.at[pl.ds(0, _SUBS * _SUB)], sem_out).wait()


def _sc_score(flat_ids, table, outv):
    mesh = plsc.VectorSubcoreMesh(core_axis_name="c", subcore_axis_name="s")
    call = functools.partial(
        pl.kernel,
        mesh=mesh,
        compiler_params=pltpu.CompilerParams(use_tc_tiling_on_sc=False, needs_layout_passes=False),
        out_type=jax.ShapeDtypeStruct((_B * _MAX_OUT,), jnp.float32),
        scratch_types=[
            pltpu.VMEM((2 * _SUBS, _SUB), jnp.int32),
            pltpu.VMEM((_SUB, _EMB), jnp.float32),
            pltpu.VMEM((_SUB, _EMB), jnp.float32),
            pltpu.VMEM((_SUB, _EMB), jnp.float32),
            pltpu.VMEM((_SUB, _EMB), jnp.float32),
            pltpu.VMEM((_BPW, _EMB), jnp.float32),
            pltpu.VMEM((2 * _SC_PAD,), jnp.float32),
            pltpu.SemaphoreType.DMA,
            pltpu.SemaphoreType.DMA,
            pltpu.SemaphoreType.DMA,
        ],
    )(_sc_score_body)
    return call(flat_ids, table, outv)


def _lstm_mlp_body(prev_ref, h_ref, c_ref, q_ref, wih_ref, whh_ref, b_ref,
                   l1_ref, l1b_ref, l2_ref, l2b_ref, out_ref):
    gates = (jnp.dot(prev_ref[...], wih_ref[...],
                     preferred_element_type=jnp.float32)
             + jnp.dot(h_ref[...], whh_ref[...],
                       preferred_element_type=jnp.float32)
             + b_ref[...])
    i_g = jax.nn.sigmoid(gates[:, :_STATE])
    f_g = jax.nn.sigmoid(gates[:, _STATE:2 * _STATE])
    g_g = jnp.tanh(gates[:, 2 * _STATE:3 * _STATE])
    o_g = jax.nn.sigmoid(gates[:, 3 * _STATE:])
    c_new = f_g * c_ref[...] + i_g * g_g
    h_new = o_g * jnp.tanh(c_new)
    sq = jnp.concatenate([h_new, q_ref[...]], axis=1)
    hid = jnp.maximum(
        jnp.dot(sq, l1_ref[...], preferred_element_type=jnp.float32)
        + l1b_ref[...], 0.0)
    out_ref[...] = jnp.maximum(
        jnp.dot(hid, l2_ref[...], preferred_element_type=jnp.float32)
        + l2b_ref[...], 0.0)


def _sample_body(flag_ref, prelim_ref, rel_ref, ent_ref, gum_ref, rnd_ref,
                 loss_ref, logits_ref, aid_ref, nent_ref, crel_ref):
    flag = flag_ref[0, 0]
    ms = jnp.where(flag != 0, rnd_ref[...], prelim_ref[...])
    ent = ent_ref[...]
    scores = jnp.where(ent == 0, jnp.float32(_NEG), ms)
    m = jnp.max(scores, axis=1, keepdims=True)
    shifted = scores - m
    lse = jnp.log(jnp.sum(jnp.exp(shifted), axis=1, keepdims=True))
    logits = shifted - lse
    z = logits + gum_ref[...]
    zmax = jnp.max(z, axis=1, keepdims=True)
    iota = lax.broadcasted_iota(jnp.int32, z.shape, 1)
    aid = jnp.min(jnp.where(z == zmax, iota, _MAX_OUT), axis=1)
    sel = iota == aid[:, None]
    loss = -jnp.sum(jnp.where(sel, logits, 0.0), axis=1)
    crel = jnp.sum(jnp.where(sel, rel_ref[...], 0), axis=1)
    nent = jnp.sum(jnp.where(sel, ent, 0), axis=1)
    logits_ref[...] = logits
    loss_ref[...] = loss[:, None]
    aid_ref[...] = aid[:, None]
    nent_ref[...] = nent[:, None]
    crel_ref[...] = crel[:, None]


_BB = 2048
_BD = 2048


def _tc_lstm_mlp(prev_emb, q_emb, state_h, state_c, wih_t, whh_t, bias,
                 l1_t, l1b, l2_t, l2b):
    nb = _B // _BB
    row = lambda i: (i, 0)
    rep = lambda i: (0, 0)
    return pl.pallas_call(
        _lstm_mlp_body,
        grid=(nb,),
        in_specs=[
            pl.BlockSpec((_BB, _EMB), row),
            pl.BlockSpec((_BB, _STATE), row),
            pl.BlockSpec((_BB, _STATE), row),
            pl.BlockSpec((_BB, _EMB), row),
            pl.BlockSpec((_EMB, 4 * _STATE), rep),
            pl.BlockSpec((_STATE, 4 * _STATE), rep),
            pl.BlockSpec((1, 4 * _STATE), rep),
            pl.BlockSpec((_STATE + _EMB, _HID), rep),
            pl.BlockSpec((1, _HID), rep),
            pl.BlockSpec((_HID, _EMB), rep),
            pl.BlockSpec((1, _EMB), rep),
        ],
        out_specs=pl.BlockSpec((_BB, _EMB), row),
        out_shape=jax.ShapeDtypeStruct((_B, _EMB), jnp.float32),
    )(prev_emb, state_h, state_c, q_emb, wih_t, whh_t, bias,
      l1_t, l1b, l2_t, l2b)


def _tc_sample(flag, prelim, rel_ids, ent_ids, gum, rnd):
    nb = _B // _BD
    row = lambda i: (i, 0)
    return pl.pallas_call(
        _sample_body,
        grid=(nb,),
        in_specs=[
            pl.BlockSpec(memory_space=pltpu.SMEM),
            pl.BlockSpec((_BD, _MAX_OUT), row),
            pl.BlockSpec((_BD, _MAX_OUT), row),
            pl.BlockSpec((_BD, _MAX_OUT), row),
            pl.BlockSpec((_BD, _MAX_OUT), row),
            pl.BlockSpec((_BD, _MAX_OUT), row),
        ],
        out_specs=[
            pl.BlockSpec((_BD, 1), row),
            pl.BlockSpec((_BD, _MAX_OUT), row),
            pl.BlockSpec((_BD, 1), row),
            pl.BlockSpec((_BD, 1), row),
            pl.BlockSpec((_BD, 1), row),
        ],
        out_shape=[
            jax.ShapeDtypeStruct((_B, 1), jnp.float32),
            jax.ShapeDtypeStruct((_B, _MAX_OUT), jnp.float32),
            jax.ShapeDtypeStruct((_B, 1), jnp.int32),
            jax.ShapeDtypeStruct((_B, 1), jnp.int32),
            jax.ShapeDtypeStruct((_B, 1), jnp.int32),
        ],
    )(flag, prelim, rel_ids, ent_ids, gum, rnd)


def kernel(prev_relation, current_entity, actions_id, queries, random,
           item_embedding, w_ih, w_hh, b_ih, b_hh, mlp_l1_w, mlp_l1_b,
           mlp_l2_w, mlp_l2_b, state_h, state_c):
    rel_ids = actions_id[:, :, 0]
    ent_ids = actions_id[:, :, 1]

    pair_idx = jnp.concatenate(
        [prev_relation, queries]).astype(jnp.int32).reshape(_NW * 8, 128)
    pair_emb = _sc_gather_pairs(pair_idx, item_embedding)
    prev_emb = pair_emb[:_B]
    q_emb = pair_emb[_B:]

    outv = _tc_lstm_mlp(
        prev_emb, q_emb, state_h, state_c,
        w_ih.T, w_hh.T, (b_ih + b_hh).reshape(1, 4 * _STATE),
        mlp_l1_w.T, mlp_l1_b.reshape(1, _HID),
        mlp_l2_w.T, mlp_l2_b.reshape(1, _EMB))

    flat_ids = rel_ids.astype(jnp.int32).reshape(_B * _MAX_OUT // _SUB, _SUB)
    model_scores = _sc_score(flat_ids, item_embedding, outv)
    model_scores = model_scores.reshape(_B, _MAX_OUT)

    gum = jax.random.gumbel(jax.random.key(42), (_B, _MAX_OUT), jnp.float32)
    rnd = jax.random.normal(jax.random.key(7), (_B, _MAX_OUT), jnp.float32)
    flag = jnp.asarray(random, jnp.int32).reshape(1, 1)

    loss, logits, aid, nent, crel = _tc_sample(
        flag, model_scores, rel_ids, ent_ids, gum, rnd)
    return (loss[:, 0], logits, aid[:, 0], nent[:, 0], crel[:, 0])


# speedup vs baseline: 3.1173x; 1.4913x over previous
import functools

import jax
import jax.numpy as jnp
from jax import lax
from jax.experimental import pallas as pl
from jax.experimental.pallas import tpu as pltpu
from jax.experimental.pallas import tpu_sc as plsc

_B = 16384
_MAX_OUT = 200
_EMB = 64
_STATE = 128
_HID = 256
_NEG = -99999.0

_NC = 2
_NS = 16
_NW = _NC * _NS
_LANES = 16

_BPW = _B // _NW
_CB = 8
_NCHUNK = _BPW // _CB
_SUB = 100
_SUBS = _CB * _MAX_OUT // _SUB
_JB = 7
_SC_PAD = _CB * _MAX_OUT + _LANES

def _sc_gather_pairs_body(idx_hbm, table_hbm, out_hbm, ids_v, rows_v, sem):
    wid = lax.axis_index("s") * _NC + lax.axis_index("c")
    pltpu.sync_copy(idx_hbm.at[pl.ds(wid * 8, 8)], ids_v)
    for sub in range(8):
        pltpu.async_copy(
            table_hbm.at[ids_v.at[sub]],
            rows_v.at[pl.ds(sub * 128, 128)],
            sem,
        ).wait()
    pltpu.sync_copy(rows_v, out_hbm.at[pl.ds(wid * 1024, 1024)])


def _sc_gather_pairs(pair_idx, table):
    mesh = plsc.VectorSubcoreMesh(core_axis_name="c", subcore_axis_name="s")
    call = functools.partial(
        pl.kernel,
        mesh=mesh,
        compiler_params=pltpu.CompilerParams(use_tc_tiling_on_sc=False, needs_layout_passes=False),
        out_type=jax.ShapeDtypeStruct((2 * _B, _EMB), jnp.float32),
        scratch_types=[
            pltpu.VMEM((8, 128), jnp.int32),
            pltpu.VMEM((1024, _EMB), jnp.float32),
            pltpu.SemaphoreType.DMA,
        ],
    )(_sc_gather_pairs_body)
    return call(pair_idx, table)


_RING = 8


def _sc_score_body(ids_hbm, table_hbm, outv_hbm, scores_hbm,
                   ids_v, rows_v, outv_v, scores_v,
                   sem_rows, sem_ids, sem_out):
    wid = lax.axis_index("s") * _NC + lax.axis_index("c")
    nsub = _BPW * _MAX_OUT // _SUB
    nblk = nsub // _SUBS
    depth = _RING - 1
    pltpu.sync_copy(outv_hbm.at[pl.ds(wid * _BPW, _BPW)], outv_v)
    lane = lax.iota(jnp.int32, _LANES)
    idrows_pw = _BPW * _MAX_OUT // _SUB

    def ids_row(s):
        return ((s // _SUBS) % 2) * _SUBS + s % _SUBS

    def fire_rows(s):
        pltpu.async_copy(table_hbm.at[ids_v.at[ids_row(s)]],
                         rows_v.at[pl.ds((s % _RING) * _SUB, _SUB)], sem_rows)

    def fire_ids(blk):
        pltpu.async_copy(
            ids_hbm.at[pl.ds(wid * idrows_pw + blk * _SUBS, _SUBS)],
            ids_v.at[pl.ds((blk % 2) * _SUBS, _SUBS)], sem_ids)

    pltpu.sync_copy(ids_hbm.at[pl.ds(wid * idrows_pw, _SUBS)],
                    ids_v.at[pl.ds(0, _SUBS)])
    fire_ids(1)
    for u in range(depth):
        fire_rows(u)

    colv = [(lane + k0) & 15 for k0 in range(_LANES)]

    def compute(s):
        par = (s // _SUBS) % 2
        base = par * _SC_PAD + (s % _SUBS) * _SUB
        rbase = (s % _RING) * _SUB
        ov = [outv_v[s // 2, pl.ds(g * _LANES, _LANES)]
              for g in range(_EMB // _LANES)]
        for jb in range(_JB):
            row_ids = rbase + jnp.minimum(jb * _LANES + lane, _SUB - 1)
            accs = [jnp.zeros((_LANES,), jnp.float32) for _ in range(4)]
            for k in range(_EMB):
                g, k0 = k // _LANES, k % _LANES
                vals = plsc.load_gather(
                    rows_v, [row_ids, colv[k0] + g * _LANES])
                mult = ov[g].at[colv[k0]].get(mode="promise_in_bounds")
                accs[k % 4] = accs[k % 4] + vals * mult
            scores_v[pl.ds(base + jb * _LANES, _LANES)] = (
                (accs[0] + accs[1]) + (accs[2] + accs[3]))

    def sub_body(s, carry):
        t = s + depth
        blk = s // _SUBS

        @pl.when(jnp.logical_and(t % _SUBS == 0, t < nsub))
        def _():
            pltpu.make_async_copy(
                ids_hbm.at[pl.ds(0, _SUBS)],
                ids_v.at[pl.ds(0, _SUBS)], sem_ids).wait()

        pltpu.make_async_copy(
            table_hbm.at[ids_v.at[0]],
            rows_v.at[pl.ds(0, _SUB)], sem_rows).wait()

        @pl.when(t < nsub)
        def _():
            fire_rows(t)

        compute(s)

        @pl.when(jnp.logical_and(jnp.logical_and(
            t % _SUBS == _SUBS - 1, s + depth < nsub), blk + 2 < nblk))
        def _():
            fire_ids(blk + 2)

        @pl.when(s % _SUBS == _SUBS - 1)
        def _():
            @pl.when(blk >= 1)
            def _():
                pltpu.make_async_copy(
                    scores_v.at[pl.ds(0, _SUBS * _SUB)],
                    scores_hbm.at[pl.ds(0, _SUBS * _SUB)],
                    sem_out).wait()
            pltpu.async_copy(
                scores_v.at[pl.ds(((s // _SUBS) % 2) * _SC_PAD,
                                  _SUBS * _SUB)],
                scores_hbm.at[pl.ds(wid * _BPW * _MAX_OUT
                                    + blk * _SUBS * _SUB, _SUBS * _SUB)],
                sem_out)
        return carry

    lax.fori_loop(0, nsub, sub_body, 0)
    pltpu.make_async_copy(
        scores_v.at[pl.ds(0, _SUBS * _SUB)],
        scores_hbm.at[pl.ds(0, _SUBS * _SUB)], sem_out).wait()


def _sc_score(flat_ids, table, outv):
    mesh = plsc.VectorSubcoreMesh(core_axis_name="c", subcore_axis_name="s")
    call = functools.partial(
        pl.kernel,
        mesh=mesh,
        compiler_params=pltpu.CompilerParams(use_tc_tiling_on_sc=False, needs_layout_passes=False),
        out_type=jax.ShapeDtypeStruct((_B * _MAX_OUT,), jnp.float32),
        scratch_types=[
            pltpu.VMEM((2 * _SUBS, _SUB), jnp.int32),
            pltpu.VMEM((_RING * _SUB, _EMB), jnp.float32),
            pltpu.VMEM((_BPW, _EMB), jnp.float32),
            pltpu.VMEM((2 * _SC_PAD,), jnp.float32),
            pltpu.SemaphoreType.DMA,
            pltpu.SemaphoreType.DMA,
            pltpu.SemaphoreType.DMA,
        ],
    )(_sc_score_body)
    return call(flat_ids, table, outv)


def _lstm_mlp_body(prev_ref, h_ref, c_ref, q_ref, wih_ref, whh_ref, b_ref,
                   l1_ref, l1b_ref, l2_ref, l2b_ref, out_ref):
    gates = (jnp.dot(prev_ref[...], wih_ref[...],
                     preferred_element_type=jnp.float32)
             + jnp.dot(h_ref[...], whh_ref[...],
                       preferred_element_type=jnp.float32)
             + b_ref[...])
    i_g = jax.nn.sigmoid(gates[:, :_STATE])
    f_g = jax.nn.sigmoid(gates[:, _STATE:2 * _STATE])
    g_g = jnp.tanh(gates[:, 2 * _STATE:3 * _STATE])
    o_g = jax.nn.sigmoid(gates[:, 3 * _STATE:])
    c_new = f_g * c_ref[...] + i_g * g_g
    h_new = o_g * jnp.tanh(c_new)
    sq = jnp.concatenate([h_new, q_ref[...]], axis=1)
    hid = jnp.maximum(
        jnp.dot(sq, l1_ref[...], preferred_element_type=jnp.float32)
        + l1b_ref[...], 0.0)
    out_ref[...] = jnp.maximum(
        jnp.dot(hid, l2_ref[...], preferred_element_type=jnp.float32)
        + l2b_ref[...], 0.0)


def _sample_body(prelim_ref, rel_ref, ent_ref, gum_ref,
                 loss_ref, logits_ref, aid_ref, nent_ref, crel_ref):
    ms = prelim_ref[...]
    ent = ent_ref[...]
    scores = jnp.where(ent == 0, jnp.float32(_NEG), ms)
    m = jnp.max(scores, axis=1, keepdims=True)
    shifted = scores - m
    lse = jnp.log(jnp.sum(jnp.exp(shifted), axis=1, keepdims=True))
    logits = shifted - lse
    z = logits + gum_ref[...]
    zmax = jnp.max(z, axis=1, keepdims=True)
    iota = lax.broadcasted_iota(jnp.int32, z.shape, 1)
    aid = jnp.min(jnp.where(z == zmax, iota, _MAX_OUT), axis=1)
    sel = iota == aid[:, None]
    loss = -jnp.sum(jnp.where(sel, logits, 0.0), axis=1)
    crel = jnp.sum(jnp.where(sel, rel_ref[...], 0), axis=1)
    nent = jnp.sum(jnp.where(sel, ent, 0), axis=1)
    logits_ref[...] = logits
    loss_ref[...] = loss[:, None]
    aid_ref[...] = aid[:, None]
    nent_ref[...] = nent[:, None]
    crel_ref[...] = crel[:, None]


_BB = 2048
_BD = 2048


def _tc_lstm_mlp(prev_emb, q_emb, state_h, state_c, wih_t, whh_t, bias,
                 l1_t, l1b, l2_t, l2b):
    nb = _B // _BB
    row = lambda i: (i, 0)
    rep = lambda i: (0, 0)
    return pl.pallas_call(
        _lstm_mlp_body,
        grid=(nb,),
        in_specs=[
            pl.BlockSpec((_BB, _EMB), row),
            pl.BlockSpec((_BB, _STATE), row),
            pl.BlockSpec((_BB, _STATE), row),
            pl.BlockSpec((_BB, _EMB), row),
            pl.BlockSpec((_EMB, 4 * _STATE), rep),
            pl.BlockSpec((_STATE, 4 * _STATE), rep),
            pl.BlockSpec((1, 4 * _STATE), rep),
            pl.BlockSpec((_STATE + _EMB, _HID), rep),
            pl.BlockSpec((1, _HID), rep),
            pl.BlockSpec((_HID, _EMB), rep),
            pl.BlockSpec((1, _EMB), rep),
        ],
        out_specs=pl.BlockSpec((_BB, _EMB), row),
        out_shape=jax.ShapeDtypeStruct((_B, _EMB), jnp.float32),
    )(prev_emb, state_h, state_c, q_emb, wih_t, whh_t, bias,
      l1_t, l1b, l2_t, l2b)


def _tc_sample(prelim, rel_ids, ent_ids, gum):
    nb = _B // _BD
    row = lambda i: (i, 0)
    return pl.pallas_call(
        _sample_body,
        grid=(nb,),
        in_specs=[
            pl.BlockSpec((_BD, _MAX_OUT), row),
            pl.BlockSpec((_BD, _MAX_OUT), row),
            pl.BlockSpec((_BD, _MAX_OUT), row),
            pl.BlockSpec((_BD, _MAX_OUT), row),
        ],
        out_specs=[
            pl.BlockSpec((_BD, 1), row),
            pl.BlockSpec((_BD, _MAX_OUT), row),
            pl.BlockSpec((_BD, 1), row),
            pl.BlockSpec((_BD, 1), row),
            pl.BlockSpec((_BD, 1), row),
        ],
        out_shape=[
            jax.ShapeDtypeStruct((_B, 1), jnp.float32),
            jax.ShapeDtypeStruct((_B, _MAX_OUT), jnp.float32),
            jax.ShapeDtypeStruct((_B, 1), jnp.int32),
            jax.ShapeDtypeStruct((_B, 1), jnp.int32),
            jax.ShapeDtypeStruct((_B, 1), jnp.int32),
        ],
    )(prelim, rel_ids, ent_ids, gum)


def kernel(prev_relation, current_entity, actions_id, queries, random,
           item_embedding, w_ih, w_hh, b_ih, b_hh, mlp_l1_w, mlp_l1_b,
           mlp_l2_w, mlp_l2_b, state_h, state_c):
    rel_ids = actions_id[:, :, 0]
    ent_ids = actions_id[:, :, 1]

    pair_idx = jnp.concatenate(
        [prev_relation, queries]).astype(jnp.int32).reshape(_NW * 8, 128)
    pair_emb = _sc_gather_pairs(pair_idx, item_embedding)
    prev_emb = pair_emb[:_B]
    q_emb = pair_emb[_B:]

    outv = _tc_lstm_mlp(
        prev_emb, q_emb, state_h, state_c,
        w_ih.T, w_hh.T, (b_ih + b_hh).reshape(1, 4 * _STATE),
        mlp_l1_w.T, mlp_l1_b.reshape(1, _HID),
        mlp_l2_w.T, mlp_l2_b.reshape(1, _EMB))

    flat_ids = rel_ids.astype(jnp.int32).reshape(_B * _MAX_OUT // _SUB, _SUB)
    model_scores = _sc_score(flat_ids, item_embedding, outv)
    model_scores = model_scores.reshape(_B, _MAX_OUT)

    gum = jax.random.gumbel(jax.random.key(42), (_B, _MAX_OUT), jnp.float32)
    prelim = lax.cond(
        jnp.asarray(random, jnp.bool_),
        lambda ms: jax.random.normal(jax.random.key(7), (_B, _MAX_OUT),
                                     jnp.float32),
        lambda ms: ms,
        model_scores)

    loss, logits, aid, nent, crel = _tc_sample(
        prelim, rel_ids, ent_ids, gum)
    return (loss[:, 0], logits, aid[:, 0], nent[:, 0], crel[:, 0])
